# baseline (speedup 1.0000x reference)
import functools

import jax
import jax.numpy as jnp
from jax import lax
from jax.experimental import pallas as pl
from jax.experimental.pallas import tpu as pltpu
from jax.experimental.pallas import tpu_sc as plsc

B, T, F = 256, 64, 39
NUM_COUNT = 13
N_CAT = 26
VOCAB = 100001
H = 32
N = B * T
CB = 128
NA = N // CB
AG = 8
IQ = 64


def _tc_prep(x_bf):
    def body(x_ref, o_ref):
        xf = x_ref[...]
        row = lax.broadcasted_iota(jnp.int32, (N_CAT, F), 0)
        col = lax.broadcasted_iota(jnp.int32, (N_CAT, F), 1)
        sel = (col == row + NUM_COUNT).astype(jnp.float32)
        ys = []
        for al in range(AG):
            xa = lax.slice(xf, (al * CB, 0), ((al + 1) * CB, F))
            ys.append(
                lax.dot_general(
                    sel, xa, (((1,), (1,)), ((), ())),
                    preferred_element_type=jnp.float32,
                )[:, None, :]
            )
        o_ref[...] = jnp.concatenate(ys, axis=1).astype(jnp.int32)

    return pl.pallas_call(
        body,
        grid=(NA // AG,),
        in_specs=[pl.BlockSpec((AG * CB, F), lambda g: (g, 0))],
        out_specs=pl.BlockSpec((N_CAT, AG, CB), lambda g: (0, g, 0)),
        out_shape=jax.ShapeDtypeStruct((N_CAT, NA, CB), jnp.int32),
    )(x_bf)


def _sc_cat_sum_t(tab_t, idx3):
    mesh = plsc.VectorSubcoreMesh(core_axis_name="c", subcore_axis_name="s")

    @functools.partial(
        pl.kernel,
        mesh=mesh,
        out_type=jax.ShapeDtypeStruct((H, N), jnp.float32),
        compiler_params=pltpu.CompilerParams(
            use_tc_tiling_on_sc=True, needs_layout_passes=False
        ),
        scratch_types=[
            pltpu.VMEM((VOCAB,), jnp.float32),
            pltpu.VMEM((IQ, CB), jnp.int32),
            pltpu.VMEM((N,), jnp.float32),
        ],
    )
    def k(tab_hbm, idx_hbm, out_hbm, row_v, idx_v, acc_v):
        h = lax.axis_index("s") * 2 + lax.axis_index("c")

        def zero(g, carry):
            acc_v[pl.ds(g * 16, 16)] = jnp.zeros((16,), jnp.float32)
            return carry

        lax.fori_loop(0, N // 16, zero, 0)

        def per_j(j, carry):
            pltpu.sync_copy(tab_hbm.at[j, h], row_v)

            def per_q(q, inner):
                pltpu.sync_copy(idx_hbm.at[j, pl.ds(q * IQ, IQ)], idx_v)

                def gath(r, c2):
                    nb = (q * IQ + r) * CB
                    for u in range(CB // 16):
                        iv = idx_v[r, pl.ds(u * 16, 16)]
                        val = plsc.load_gather(row_v, [iv])
                        nu = nb + u * 16
                        acc_v[pl.ds(nu, 16)] = acc_v[pl.ds(nu, 16)] + val
                    return c2

                lax.fori_loop(0, IQ, gath, 0)
                return inner

            lax.fori_loop(0, NA // IQ, per_q, 0)
            return carry

        lax.fori_loop(0, N_CAT, per_j, 0)
        pltpu.sync_copy(acc_v, out_hbm.at[h])

    return k(tab_t, idx3)


def _tc_finish(x_bf, num_embeddings, cat_t):
    def body(x_ref, emb_ref, cat_ref, o_ref):
        xf = x_ref[...]
        e = emb_ref[0]
        embp = jnp.concatenate(
            [e, jnp.zeros((F - NUM_COUNT, H), jnp.float32)], axis=0
        )
        m = jnp.dot(xf, embp, preferred_element_type=jnp.float32)
        c = cat_ref[...]
        row = lax.broadcasted_iota(jnp.int32, (H, H), 0)
        col = lax.broadcasted_iota(jnp.int32, (H, H), 1)
        eye = (row == col).astype(jnp.float32)
        y = lax.dot_general(
            c, eye, (((0,), (0,)), ((), ())),
            preferred_element_type=jnp.float32,
        )
        o_ref[...] = (m + y).reshape(CB // T, T, H)

    return pl.pallas_call(
        body,
        grid=(NA,),
        in_specs=[
            pl.BlockSpec((CB, F), lambda a: (a, 0)),
            pl.BlockSpec((1, NUM_COUNT, H), lambda a: (0, 0, 0)),
            pl.BlockSpec((H, CB), lambda a: (0, a)),
        ],
        out_specs=pl.BlockSpec((CB // T, T, H), lambda a: (a, 0, 0)),
        out_shape=jax.ShapeDtypeStruct((B, T, H), jnp.float32),
    )(x_bf, num_embeddings, cat_t)


def kernel(x_bt_f, tables, num_embeddings):
    x_bf = x_bt_f.reshape(N, F)
    tab_t = tables.transpose(0, 2, 1)
    idx3 = _tc_prep(x_bf)
    cat_t = _sc_cat_sum_t(tab_t, idx3)
    return _tc_finish(x_bf, num_embeddings, cat_t)
